# Initial kernel scaffold; baseline (speedup 1.0000x reference)
import jax
import jax.numpy as jnp
from jax.experimental import pallas as pl


def kernel(x, edge_index, W1, b1, W2, b2):
    raise NotImplementedError("write your pallas kernel here")


# speedup vs baseline: 10.2588x; 10.2588x over previous
import functools

import jax
import jax.numpy as jnp
from jax import lax
from jax.experimental import pallas as pl
from jax.experimental.pallas import tpu as pltpu
from jax.experimental.pallas import tpu_sc as plsc

N = 10000
D = 128
E = 320000

NW = 32
B = 128
NP = 10240
EP = ((E + NW * B - 1) // (NW * B)) * (NW * B)
EPW = EP // NW
NBATCH = EPW // B
ROWS_PER_TILE = NP // 16

_mesh = plsc.VectorSubcoreMesh(core_axis_name="c", subcore_axis_name="s")


@functools.partial(
    pl.kernel,
    mesh=_mesh,
    out_type=jax.ShapeDtypeStruct((2 * NP,), jnp.float32),
    scratch_types=[
        pltpu.VMEM((B,), jnp.int32),
        pltpu.VMEM((B,), jnp.float32),
        pltpu.VMEM_SHARED((NP,), jnp.float32),
    ],
)
def _sc_deg(dst_hbm, zeros_hbm, out_hbm, dst_v, ones_v, deg_sh):
    c = lax.axis_index("c")
    s = lax.axis_index("s")
    for i in range(B // 16):
        ones_v[pl.ds(i * 16, 16)] = jnp.ones((16,), jnp.float32)
    pltpu.sync_copy(zeros_hbm.at[pl.ds(0, ROWS_PER_TILE)],
                    deg_sh.at[pl.ds(s * ROWS_PER_TILE, ROWS_PER_TILE)])
    plsc.subcore_barrier()
    base = (c * 16 + s) * EPW

    def body(i, carry):
        off = pl.multiple_of(base + i * B, B)
        pltpu.sync_copy(dst_hbm.at[pl.ds(off, B)], dst_v)
        pltpu.sync_copy(ones_v, deg_sh.at[dst_v], add=True)
        return carry

    lax.fori_loop(0, NBATCH, body, 0)
    plsc.subcore_barrier()
    out_off = pl.multiple_of(c * NP + s * ROWS_PER_TILE, ROWS_PER_TILE)
    pltpu.sync_copy(deg_sh.at[pl.ds(s * ROWS_PER_TILE, ROWS_PER_TILE)],
                    out_hbm.at[pl.ds(out_off, ROWS_PER_TILE)])


@functools.partial(
    pl.kernel,
    mesh=_mesh,
    out_type=jax.ShapeDtypeStruct((2 * NP, D), jnp.float32),
    scratch_types=[
        pltpu.VMEM((B,), jnp.int32),
        pltpu.VMEM((B,), jnp.int32),
        pltpu.VMEM((B, D), jnp.float32),
        pltpu.VMEM_SHARED((NP, D), jnp.float32),
        pltpu.SemaphoreType.DMA,
    ],
)
def _sc_agg(tab_hbm, src_hbm, dst_hbm, zrows_hbm, out_hbm,
            src_v, dst_v, rows_v, acc_sh, sem):
    c = lax.axis_index("c")
    s = lax.axis_index("s")
    pltpu.sync_copy(zrows_hbm.at[pl.ds(0, ROWS_PER_TILE)],
                    acc_sh.at[pl.ds(s * ROWS_PER_TILE, ROWS_PER_TILE)])
    plsc.subcore_barrier()
    base = (c * 16 + s) * EPW

    def body(i, carry):
        off = pl.multiple_of(base + i * B, B)
        pltpu.sync_copy(src_hbm.at[pl.ds(off, B)], src_v)
        pltpu.sync_copy(dst_hbm.at[pl.ds(off, B)], dst_v)
        pltpu.async_copy(tab_hbm.at[src_v], rows_v, sem).wait()
        pltpu.sync_copy(rows_v, acc_sh.at[dst_v], add=True)
        return carry

    lax.fori_loop(0, NBATCH, body, 0)
    plsc.subcore_barrier()
    out_off = pl.multiple_of(c * NP + s * ROWS_PER_TILE, ROWS_PER_TILE)
    pltpu.sync_copy(acc_sh.at[pl.ds(s * ROWS_PER_TILE, ROWS_PER_TILE)],
                    out_hbm.at[pl.ds(out_off, ROWS_PER_TILE)])


_BM = 1024


def _tc_layer1(x_pad, W1, d0, d1):
    def body(x_ref, w_ref, d0_ref, d1_ref, g_ref, dis_ref):
        deg = d0_ref[...] + d1_ref[...] + 1.0
        dis = lax.rsqrt(deg)
        h = jnp.dot(x_ref[...], w_ref[...], preferred_element_type=jnp.float32)
        g_ref[...] = h * dis
        dis_ref[...] = dis

    return pl.pallas_call(
        body,
        grid=(NP // _BM,),
        in_specs=[
            pl.BlockSpec((_BM, D), lambda m: (m, 0)),
            pl.BlockSpec((D, D), lambda m: (0, 0)),
            pl.BlockSpec((_BM, 1), lambda m: (m, 0)),
            pl.BlockSpec((_BM, 1), lambda m: (m, 0)),
        ],
        out_specs=[
            pl.BlockSpec((_BM, D), lambda m: (m, 0)),
            pl.BlockSpec((_BM, 1), lambda m: (m, 0)),
        ],
        out_shape=[
            jax.ShapeDtypeStruct((NP, D), jnp.float32),
            jax.ShapeDtypeStruct((NP, 1), jnp.float32),
        ],
    )(x_pad, W1, d0, d1)


def _tc_layer2(p0, p1, g1, dis, b1, W2):
    def body(p0_ref, p1_ref, g_ref, dis_ref, b_ref, w_ref, g2_ref):
        dis = dis_ref[...]
        pre = dis * (p0_ref[...] + p1_ref[...] + g_ref[...]) + b_ref[...]
        h = jnp.maximum(pre, 0.0)
        g2_ref[...] = dis * jnp.dot(h, w_ref[...],
                                    preferred_element_type=jnp.float32)

    return pl.pallas_call(
        body,
        grid=(NP // _BM,),
        in_specs=[
            pl.BlockSpec((_BM, D), lambda m: (m, 0)),
            pl.BlockSpec((_BM, D), lambda m: (m, 0)),
            pl.BlockSpec((_BM, D), lambda m: (m, 0)),
            pl.BlockSpec((_BM, 1), lambda m: (m, 0)),
            pl.BlockSpec((1, D), lambda m: (0, 0)),
            pl.BlockSpec((D, D), lambda m: (0, 0)),
        ],
        out_specs=pl.BlockSpec((_BM, D), lambda m: (m, 0)),
        out_shape=jax.ShapeDtypeStruct((NP, D), jnp.float32),
    )(p0, p1, g1, dis, b1, W2)


def _tc_epilogue(q0, q1, g2, dis, b2):
    def body(q0_ref, q1_ref, g_ref, dis_ref, b_ref, o_ref):
        o_ref[...] = (dis_ref[...] * (q0_ref[...] + q1_ref[...] + g_ref[...])
                      + b_ref[...])

    return pl.pallas_call(
        body,
        grid=(NP // _BM,),
        in_specs=[
            pl.BlockSpec((_BM, D), lambda m: (m, 0)),
            pl.BlockSpec((_BM, D), lambda m: (m, 0)),
            pl.BlockSpec((_BM, D), lambda m: (m, 0)),
            pl.BlockSpec((_BM, 1), lambda m: (m, 0)),
            pl.BlockSpec((1, D), lambda m: (0, 0)),
        ],
        out_specs=pl.BlockSpec((_BM, D), lambda m: (m, 0)),
        out_shape=jax.ShapeDtypeStruct((NP, D), jnp.float32),
    )(q0, q1, g2, dis, b2)


def kernel(x, edge_index, W1, b1, W2, b2):
    pad_e = EP - E
    src = jnp.concatenate([edge_index[0], jnp.zeros((pad_e,), jnp.int32)])
    dst = jnp.concatenate([edge_index[1], jnp.full((pad_e,), N, jnp.int32)])
    x_pad = jnp.pad(x, ((0, NP - N), (0, 0)))
    zvec = jnp.zeros((ROWS_PER_TILE,), jnp.float32)
    zrows = jnp.zeros((ROWS_PER_TILE, D), jnp.float32)

    degp = _sc_deg(dst, zvec)
    d0 = degp[:NP].reshape(NP, 1)
    d1 = degp[NP:].reshape(NP, 1)

    g1, dis = _tc_layer1(x_pad, W1, d0, d1)

    agg1 = _sc_agg(g1, src, dst, zrows)
    g2 = _tc_layer2(agg1[:NP], agg1[NP:], g1, dis, b1.reshape(1, D), W2)

    agg2 = _sc_agg(g2, src, dst, zrows)
    out = _tc_epilogue(agg2[:NP], agg2[NP:], g2, dis, b2.reshape(1, D))
    return out[:N]
